# baseline (speedup 1.0000x reference)
import functools

import jax
import jax.numpy as jnp
from jax import lax
from jax.experimental import pallas as pl
from jax.experimental.pallas import tpu as pltpu
from jax.experimental.pallas import tpu_sc as plsc

B, L, H = 1024, 200, 64
N = B * L
EPS = 1e-05

NC, NS, LANES = 2, 16, 16
NW = NC * NS
LPAD = 256
ROWS_W = B // NW
GROUPS = L // LANES
TAIL = L - GROUPS * LANES
SPLIT = 96
HREG = H // LANES
WROW = 2 * H
MAXLEN, TYPE_VOCAB = 200, 2
NPS = MAXLEN * TYPE_VOCAB


def _rsqrt(v):
    i = lax.bitcast_convert_type(v, jnp.int32)
    i = jnp.int32(0x5F3759DF) - lax.shift_right_logical(i, 1)
    y = lax.bitcast_convert_type(i, jnp.float32)
    half = v * 0.5
    for _ in range(3):
        y = y * (1.5 - half * y * y)
    return y


_mesh = plsc.VectorSubcoreMesh(core_axis_name="c", subcore_axis_name="s")


@functools.partial(
    pl.kernel,
    mesh=_mesh,
    out_type=jax.ShapeDtypeStruct((N * H,), jnp.float32),
    compiler_params=pltpu.CompilerParams(
        needs_layout_passes=False, use_tc_tiling_on_sc=False),
    scratch_types=[
        pltpu.VMEM((LPAD,), jnp.int32),
        pltpu.VMEM((LPAD,), jnp.int32),
        pltpu.VMEM((LPAD,), jnp.int32),
        pltpu.VMEM((LPAD,), jnp.int32),
        pltpu.VMEM((LPAD,), jnp.int32),
        pltpu.VMEM((LPAD,), jnp.int32),
        pltpu.VMEM((L, WROW), jnp.float32),
        pltpu.VMEM((L, WROW), jnp.float32),
        pltpu.VMEM((L * H,), jnp.float32),
        pltpu.VMEM((L * H,), jnp.float32),
        pltpu.VMEM((TYPE_VOCAB * H,), jnp.float32),
        pltpu.VMEM((NPS * H,), jnp.float32),
        pltpu.VMEM((H,), jnp.float32),
        pltpu.VMEM((H,), jnp.float32),
        pltpu.SemaphoreType.DMA,
        pltpu.SemaphoreType.DMA,
        pltpu.SemaphoreType.DMA,
        pltpu.SemaphoreType.DMA,
        pltpu.SemaphoreType.DMA,
        pltpu.SemaphoreType.DMA,
    ],
)
def _sc_embed(xhi_hbm, xoff_hbm, psoff_hbm, word_hbm, posw_hbm, sentw_hbm,
              gamma_hbm, beta_hbm, out_hbm,
              ihi0, ihi1, ioff0, ioff1, ips0, ips1, rows0, rows1,
              obuf0, obuf1, sentw_v, ps_v, g_v, b_v,
              isem0, isem1, gsem0, gsem1, osem0, osem1):
    wid = lax.axis_index("s") * NC + lax.axis_index("c")
    ihi = (ihi0, ihi1)
    ioff = (ioff0, ioff1)
    ips = (ips0, ips1)
    rows = (rows0, rows1)
    obuf = (obuf0, obuf1)
    isem = (isem0, isem1)
    gsem = (gsem0, gsem1)
    osem = (osem0, osem1)
    row0 = wid * ROWS_W

    pltpu.sync_copy(posw_hbm, obuf0)
    pltpu.sync_copy(sentw_hbm, sentw_v)
    pltpu.sync_copy(gamma_hbm, g_v)
    pltpu.sync_copy(beta_hbm, b_v)

    g_regs = [g_v[pl.ds(j * LANES, LANES)] for j in range(HREG)]
    b_regs = [b_v[pl.ds(j * LANES, LANES)] for j in range(HREG)]

    def ps_body(p, carry):
        for s in range(TYPE_VOCAB):
            base = (p * TYPE_VOCAB + s) * H
            for j in range(HREG):
                ps_v[pl.ds(base + j * LANES, LANES)] = \
                    obuf0[pl.ds(p * H + j * LANES, LANES)] + \
                    sentw_v[pl.ds(s * H + j * LANES, LANES)]
        return carry

    lax.fori_loop(0, MAXLEN, ps_body, 0)

    def issue_idx(r, b):
        sl = pl.ds((row0 + r) * LPAD, LPAD)
        pltpu.async_copy(xhi_hbm.at[sl], ihi[b], isem[b])
        pltpu.async_copy(xoff_hbm.at[sl], ioff[b], isem[b])
        pltpu.async_copy(psoff_hbm.at[sl], ips[b], isem[b])

    def wait_idx(r, b):
        sl = pl.ds((row0 + r) * LPAD, LPAD)
        pltpu.make_async_copy(xhi_hbm.at[sl], ihi[b], isem[b]).wait()
        pltpu.make_async_copy(xoff_hbm.at[sl], ioff[b], isem[b]).wait()
        pltpu.make_async_copy(psoff_hbm.at[sl], ips[b], isem[b]).wait()

    def issue_gather(b):
        pltpu.async_copy(word_hbm.at[ihi[b].at[pl.ds(0, SPLIT)]],
                         rows[b].at[pl.ds(0, SPLIT)], gsem[b])
        pltpu.async_copy(word_hbm.at[ihi[b].at[pl.ds(SPLIT, L - SPLIT)]],
                         rows[b].at[pl.ds(SPLIT, L - SPLIT)], gsem[b])

    def wait_gather(b):
        pltpu.make_async_copy(word_hbm.at[ihi[b].at[pl.ds(0, SPLIT)]],
                              rows[b].at[pl.ds(0, SPLIT)], gsem[b]).wait()
        pltpu.make_async_copy(word_hbm.at[ihi[b].at[pl.ds(SPLIT, L - SPLIT)]],
                              rows[b].at[pl.ds(SPLIT, L - SPLIT)],
                              gsem[b]).wait()

    def out_slice(r):
        return out_hbm.at[pl.ds((row0 + r) * L * H, L * H)]

    def compute_row(b):
        def group_body(g, n_tok, carry):
            offv = ioff[b][pl.ds(g * LANES, LANES)]
            psv = ips[b][pl.ds(g * LANES, LANES)]
            for tt in range(n_tok):
                t = g * LANES + tt
                off = offv[tt]
                psb = psv[tt]
                acc = []
                for j in range(HREG):
                    w = rows[b][t, pl.ds(off + j * LANES, LANES)]
                    p = ps_v[pl.ds(psb + j * LANES, LANES)]
                    acc.append(w + p)
                tot = (acc[0] + acc[1]) + (acc[2] + acc[3])
                sq = (acc[0] * acc[0] + acc[1] * acc[1]) + \
                     (acc[2] * acc[2] + acc[3] * acc[3])
                s1 = lax.broadcast_in_dim(jnp.sum(tot), (LANES,), ())
                s2 = lax.broadcast_in_dim(jnp.sum(sq), (LANES,), ())
                mean = s1 * (1.0 / H)
                ms = s2 * (1.0 / H)
                inv = _rsqrt(ms - mean * mean + EPS)
                minv = mean * inv
                for j in range(HREG):
                    o = (acc[j] * inv - minv) * g_regs[j] + b_regs[j]
                    obuf[b][pl.ds(t * H + j * LANES, LANES)] = o
            return carry

        lax.fori_loop(0, GROUPS,
                      lambda g, cy: group_body(g, LANES, cy), 0)
        group_body(GROUPS, TAIL, 0)

    issue_idx(0, 0)
    issue_idx(1, 1)
    wait_idx(0, 0)
    issue_gather(0)

    def pair_body(i, carry):
        for bb in range(2):
            r = i * 2 + bb
            wait_gather(bb)

            @pl.when(r + 1 < ROWS_W)
            def _():
                wait_idx(r + 1, 1 - bb)
                issue_gather(1 - bb)

            @pl.when(i >= 1)
            def _():
                pltpu.make_async_copy(obuf[bb], out_slice(r - 2),
                                      osem[bb]).wait()

            compute_row(bb)
            pltpu.async_copy(obuf[bb], out_slice(r), osem[bb])

            @pl.when(i < ROWS_W // 2 - 1)
            def _():
                issue_idx(r + 2, bb)
        return carry

    lax.fori_loop(0, ROWS_W // 2, pair_body, 0)
    pltpu.make_async_copy(obuf[0], out_slice(ROWS_W - 2), osem[0]).wait()
    pltpu.make_async_copy(obuf[1], out_slice(ROWS_W - 1), osem[1]).wait()


VOCAB = 1000000
NCHUNK = (VOCAB + WROW - 1) // WROW
PROWS = NCHUNK * H
_TAIL0 = (NCHUNK - 1) * WROW


def _repack_body(m_ref, o_ref):
    m = m_ref[...]
    o_ref[:, 0:H] = jnp.transpose(m[:, 0:H])
    o_ref[:, H:WROW] = jnp.transpose(m[:, H:WROW])


_repack = pl.pallas_call(
    _repack_body,
    grid=(NCHUNK,),
    in_specs=[pl.BlockSpec((H, WROW), lambda i: (0, i))],
    out_specs=pl.BlockSpec((H, WROW), lambda i: (i, 0)),
    out_shape=jax.ShapeDtypeStruct((PROWS, WROW), jnp.float32),
)


def kernel(x, pos_ids, sent_ids, word_W, pos_W, sent_W, gamma, beta):
    x = x.astype(jnp.int32)
    pos_ids = pos_ids.astype(jnp.int32)
    sent_ids = sent_ids.astype(jnp.int32)
    pad = ((0, 0), (0, LPAD - L))
    hi = (x // WROW) * H + (x % H)
    off = ((x % WROW) // H) * H
    tail = x >= _TAIL0
    xhi = jnp.pad(jnp.where(tail, (_TAIL0 // 2) + (x - _TAIL0), hi), pad)
    xoff = jnp.pad(jnp.where(tail, 0, off), pad)
    psoff = jnp.pad((pos_ids * TYPE_VOCAB + sent_ids) * H, pad)
    wpacked = _repack(word_W.T)
    out = _sc_embed(xhi.reshape(B * LPAD), xoff.reshape(B * LPAD),
                    psoff.reshape(B * LPAD), wpacked,
                    pos_W.reshape(MAXLEN * H), sent_W.reshape(TYPE_VOCAB * H),
                    gamma, beta)
    return out.reshape(B, L, H)


# speedup vs baseline: 4.5989x; 4.5989x over previous
import functools

import jax
import jax.numpy as jnp
from jax import lax
from jax.experimental import pallas as pl
from jax.experimental.pallas import tpu as pltpu
from jax.experimental.pallas import tpu_sc as plsc

B, L, H = 1024, 200, 64
N = B * L
EPS = 1e-05

NC, NS, LANES = 2, 16, 16
NW = NC * NS
LPAD = 256
ROWS_W = B // NW
GROUPS = L // LANES
TAIL = L - GROUPS * LANES
SPLIT = 96
HREG = H // LANES
MAXLEN, TYPE_VOCAB = 200, 2
NPS = MAXLEN * TYPE_VOCAB


def _rsqrt(v):
    i = lax.bitcast_convert_type(v, jnp.int32)
    i = jnp.int32(0x5F3759DF) - lax.shift_right_logical(i, 1)
    y = lax.bitcast_convert_type(i, jnp.float32)
    half = v * 0.5
    for _ in range(3):
        y = y * (1.5 - half * y * y)
    return y


_mesh = plsc.VectorSubcoreMesh(core_axis_name="c", subcore_axis_name="s")


@functools.partial(
    pl.kernel,
    mesh=_mesh,
    out_type=jax.ShapeDtypeStruct((N * H,), jnp.float32),
    compiler_params=pltpu.CompilerParams(
        needs_layout_passes=False, use_tc_tiling_on_sc=False),
    scratch_types=[
        pltpu.VMEM((ROWS_W * LPAD,), jnp.int32),
        pltpu.VMEM((ROWS_W * LPAD,), jnp.int32),
        pltpu.VMEM((ROWS_W * LPAD,), jnp.int32),
        pltpu.VMEM((L, H), jnp.float32),
        pltpu.VMEM((L, H), jnp.float32),
        pltpu.VMEM((L * H,), jnp.float32),
        pltpu.VMEM((L * H,), jnp.float32),
        pltpu.VMEM((TYPE_VOCAB, H), jnp.float32),
        pltpu.VMEM((NPS * H,), jnp.float32),
        pltpu.VMEM((H,), jnp.float32),
        pltpu.VMEM((H,), jnp.float32),
        pltpu.SemaphoreType.DMA,
        pltpu.SemaphoreType.DMA,
        pltpu.SemaphoreType.DMA,
        pltpu.SemaphoreType.DMA,
    ],
)
def _sc_embed(x_hbm, pos_hbm, sent_hbm, word_hbm, posw_hbm, sentw_hbm,
              gamma_hbm, beta_hbm, out_hbm,
              idx_w, idx_p, idx_s, rows0, rows1, obuf0, obuf1,
              sentw_v, ps_v, g_v, b_v,
              gsem0, gsem1, osem0, osem1):
    wid = lax.axis_index("s") * NC + lax.axis_index("c")
    rows = (rows0, rows1)
    obuf = (obuf0, obuf1)
    gsem = (gsem0, gsem1)
    osem = (osem0, osem1)

    pltpu.sync_copy(x_hbm.at[pl.ds(wid * ROWS_W * LPAD, ROWS_W * LPAD)], idx_w)
    pltpu.sync_copy(pos_hbm.at[pl.ds(wid * ROWS_W * LPAD, ROWS_W * LPAD)],
                    idx_p)
    pltpu.sync_copy(sent_hbm.at[pl.ds(wid * ROWS_W * LPAD, ROWS_W * LPAD)],
                    idx_s)
    pltpu.sync_copy(posw_hbm, rows0)
    pltpu.sync_copy(sentw_hbm, sentw_v)
    pltpu.sync_copy(gamma_hbm, g_v)
    pltpu.sync_copy(beta_hbm, b_v)

    g_regs = [g_v[pl.ds(j * LANES, LANES)] for j in range(HREG)]
    b_regs = [b_v[pl.ds(j * LANES, LANES)] for j in range(HREG)]

    def ps_body(p, carry):
        for s in range(TYPE_VOCAB):
            base = (p * TYPE_VOCAB + s) * H
            for j in range(HREG):
                sl = pl.ds(j * LANES, LANES)
                ps_v[pl.ds(base + j * LANES, LANES)] = \
                    rows0[p, sl] + sentw_v[s, sl]
        return carry

    lax.fori_loop(0, MAXLEN, ps_body, 0)

    def issue_gather(r, b):
        pltpu.async_copy(
            word_hbm.at[idx_w.at[pl.ds(r * LPAD, SPLIT)]],
            rows[b].at[pl.ds(0, SPLIT)], gsem[b])
        pltpu.async_copy(
            word_hbm.at[idx_w.at[pl.ds(r * LPAD + SPLIT, L - SPLIT)]],
            rows[b].at[pl.ds(SPLIT, L - SPLIT)], gsem[b])

    def wait_gather(r, b):
        pltpu.make_async_copy(
            word_hbm.at[idx_w.at[pl.ds(r * LPAD, SPLIT)]],
            rows[b].at[pl.ds(0, SPLIT)], gsem[b]).wait()
        pltpu.make_async_copy(
            word_hbm.at[idx_w.at[pl.ds(r * LPAD + SPLIT, L - SPLIT)]],
            rows[b].at[pl.ds(SPLIT, L - SPLIT)], gsem[b]).wait()

    def out_slice(r):
        return out_hbm.at[pl.ds((wid * ROWS_W + r) * L * H, L * H)]

    def compute_row(r, b):
        def group_body(g, n_tok, carry):
            pv = idx_p[pl.ds(r * LPAD + g * LANES, LANES)]
            sv = idx_s[pl.ds(r * LPAD + g * LANES, LANES)]
            ps_base = (pv * TYPE_VOCAB + sv) * H
            for tt in range(n_tok):
                t = g * LANES + tt
                base = ps_base[tt]
                acc = []
                for j in range(HREG):
                    w = rows[b][t, pl.ds(j * LANES, LANES)]
                    p = ps_v[pl.ds(base + j * LANES, LANES)]
                    acc.append(w + p)
                tot = (acc[0] + acc[1]) + (acc[2] + acc[3])
                sq = (acc[0] * acc[0] + acc[1] * acc[1]) + \
                     (acc[2] * acc[2] + acc[3] * acc[3])
                s1 = lax.broadcast_in_dim(jnp.sum(tot), (LANES,), ())
                s2 = lax.broadcast_in_dim(jnp.sum(sq), (LANES,), ())
                mean = s1 * (1.0 / H)
                ms = s2 * (1.0 / H)
                inv = _rsqrt(ms - mean * mean + EPS)
                minv = mean * inv
                for j in range(HREG):
                    o = (acc[j] * inv - minv) * g_regs[j] + b_regs[j]
                    obuf[b][pl.ds(t * H + j * LANES, LANES)] = o
            return carry

        lax.fori_loop(0, GROUPS,
                      lambda g, cy: group_body(g, LANES, cy), 0)
        group_body(GROUPS, TAIL, 0)

    issue_gather(0, 0)

    def pair_body(i, carry):
        for bb in range(2):
            r = i * 2 + bb
            wait_gather(r, bb)
            if bb == 0:
                issue_gather(r + 1, 1)
            else:
                @pl.when(i < ROWS_W // 2 - 1)
                def _():
                    issue_gather(r + 1, 0)

            @pl.when(i >= 1)
            def _():
                pltpu.make_async_copy(obuf[bb], out_slice(r - 2),
                                      osem[bb]).wait()

            compute_row(r, bb)
            pltpu.async_copy(obuf[bb], out_slice(r), osem[bb])
        return carry

    lax.fori_loop(0, ROWS_W // 2, pair_body, 0)
    pltpu.make_async_copy(obuf[0], out_slice(ROWS_W - 2), osem[0]).wait()
    pltpu.make_async_copy(obuf[1], out_slice(ROWS_W - 1), osem[1]).wait()


def kernel(x, pos_ids, sent_ids, word_W, pos_W, sent_W, gamma, beta):
    pad = ((0, 0), (0, LPAD - L))
    xp = jnp.pad(x.astype(jnp.int32), pad).reshape(B * LPAD)
    pp = jnp.pad(pos_ids.astype(jnp.int32), pad).reshape(B * LPAD)
    sp = jnp.pad(sent_ids.astype(jnp.int32), pad).reshape(B * LPAD)
    out = _sc_embed(xp, pp, sp, word_W, pos_W, sent_W, gamma, beta)
    return out.reshape(B, L, H)
